# baseline (speedup 1.0000x reference)
import jax
import jax.numpy as jnp
from jax import lax
from jax.experimental import pallas as pl
from jax.experimental.pallas import tpu as pltpu
from jax.experimental.pallas import tpu_sc as plsc

NUM_CORES = 2
NUM_SUBCORES = 16
LANES = 16

CHUNK = 8
NBUF = 3

R0 = 576
R1 = 448


def _body(x_hbm, spans_hbm, pe_hbm, out_hbm, idx_v, *scratch):
    x_bufs = scratch[:NBUF]
    pe_bufs = scratch[NBUF:2 * NBUF]
    gsems = scratch[2 * NBUF:3 * NBUF]
    xsems = scratch[3 * NBUF:4 * NBUF]
    osems = scratch[4 * NBUF:5 * NBUF]

    batch, seq, hidden = x_hbm.shape

    cid = lax.axis_index("c")
    tid = lax.axis_index("s")
    is0 = cid == 0
    rows_per_w = jnp.where(is0, R0, R1)
    wbase = jnp.where(is0, tid * R0, NUM_SUBCORES * R0 + tid * R1)
    n_chunks = rows_per_w // CHUNK
    pe2 = pe_hbm.at[0]

    def idx_slice(ci):
        return idx_v.at[pl.ds(pl.multiple_of(ci * CHUNK, CHUNK), CHUNK)]

    def row_slice(hbm, ci):
        flat = wbase + ci * CHUNK
        return hbm.at[flat // seq, pl.ds(flat % seq, CHUNK)]

    def drain(dst, sem):
        pltpu.make_async_copy(x_hbm.at[0, pl.ds(0, CHUNK)], dst, sem).wait()

    @pl.when(is0)
    def _():
        pltpu.sync_copy(
            spans_hbm.at[pl.ds(pl.multiple_of(wbase, CHUNK), R0)],
            idx_v.at[pl.ds(0, R0)])

    @pl.when(jnp.logical_not(is0))
    def _():
        pltpu.sync_copy(
            spans_hbm.at[pl.ds(pl.multiple_of(wbase, CHUNK), R1)],
            idx_v.at[pl.ds(0, R1)])

    for s in range(NBUF):
        pltpu.async_copy(pe2.at[idx_slice(s)], pe_bufs[s], gsems[s])
    for s in range(NBUF - 1):
        pltpu.async_copy(row_slice(x_hbm, s), x_bufs[s], xsems[s])

    n_groups = (n_chunks + NBUF) // NBUF

    def group_body(g, carry):
        for b in range(NBUF):
            ci = g * NBUF + b

            @pl.when(ci < n_chunks)
            def _():
                drain(pe_bufs[b], gsems[b])
                drain(x_bufs[b], xsems[b])
                x_b, pe_b = x_bufs[b], pe_bufs[b]
                for r in range(CHUNK):
                    @plsc.parallel_loop(0, hidden, step=LANES, unroll=16)
                    def _(i):
                        seg = pl.ds(pl.multiple_of(i, LANES), LANES)
                        plsc.addupdate(x_b.at[r, seg], pe_b[r, seg])
                pltpu.async_copy(x_b, row_slice(out_hbm, ci), osems[b])
                @pl.when(ci + NBUF < n_chunks)
                def _():
                    pltpu.async_copy(
                        pe2.at[idx_slice(ci + NBUF)], pe_b, gsems[b])

            d = (b + 2) % NBUF

            @pl.when(jnp.logical_and(ci >= 1, ci - 1 < n_chunks))
            def _():
                drain(x_bufs[d], osems[d])

            @pl.when(ci + 2 < n_chunks)
            def _():
                pltpu.async_copy(
                    row_slice(x_hbm, ci + 2), x_bufs[d], xsems[d])
        return carry

    lax.fori_loop(0, n_groups, group_body, 0)


def kernel(x, spans, pe):
    batch, seq, hidden = x.shape
    assert NUM_SUBCORES * (R0 + R1) == batch * seq
    mesh = plsc.VectorSubcoreMesh(
        core_axis_name="c", subcore_axis_name="s",
        num_cores=NUM_CORES, num_subcores=NUM_SUBCORES)
    return pl.kernel(
        _body,
        out_type=jax.ShapeDtypeStruct((batch, seq, hidden), jnp.float32),
        mesh=mesh,
        scratch_types=(
            [pltpu.VMEM((R0,), jnp.int32)]
            + [pltpu.VMEM((CHUNK, hidden), jnp.float32)] * (2 * NBUF)
            + [pltpu.SemaphoreType.DMA] * (3 * NBUF)
        ),
    )(x, spans.reshape(batch * seq).astype(jnp.int32), pe)


# speedup vs baseline: 1.0005x; 1.0005x over previous
import jax
import jax.numpy as jnp
from jax import lax
from jax.experimental import pallas as pl
from jax.experimental.pallas import tpu as pltpu
from jax.experimental.pallas import tpu_sc as plsc

NUM_CORES = 2
NUM_SUBCORES = 16
LANES = 16

CHUNK = 8
NBUF = 3

R0 = 448
R1 = 576


def _body(x_hbm, spans_hbm, pe_hbm, out_hbm, idx_v, *scratch):
    x_bufs = scratch[:NBUF]
    pe_bufs = scratch[NBUF:2 * NBUF]
    gsems = scratch[2 * NBUF:3 * NBUF]
    xsems = scratch[3 * NBUF:4 * NBUF]
    osems = scratch[4 * NBUF:5 * NBUF]

    batch, seq, hidden = x_hbm.shape

    cid = lax.axis_index("c")
    tid = lax.axis_index("s")
    is0 = cid == 0
    rows_per_w = jnp.where(is0, R0, R1)
    wbase = jnp.where(is0, tid * R0, NUM_SUBCORES * R0 + tid * R1)
    n_chunks = rows_per_w // CHUNK
    pe2 = pe_hbm.at[0]

    def idx_slice(ci):
        return idx_v.at[pl.ds(pl.multiple_of(ci * CHUNK, CHUNK), CHUNK)]

    def row_slice(hbm, ci):
        flat = wbase + ci * CHUNK
        return hbm.at[flat // seq, pl.ds(flat % seq, CHUNK)]

    def drain(dst, sem):
        pltpu.make_async_copy(x_hbm.at[0, pl.ds(0, CHUNK)], dst, sem).wait()

    @pl.when(is0)
    def _():
        pltpu.sync_copy(
            spans_hbm.at[pl.ds(pl.multiple_of(wbase, CHUNK), R0)],
            idx_v.at[pl.ds(0, R0)])

    @pl.when(jnp.logical_not(is0))
    def _():
        pltpu.sync_copy(
            spans_hbm.at[pl.ds(pl.multiple_of(wbase, CHUNK), R1)],
            idx_v.at[pl.ds(0, R1)])

    for s in range(NBUF):
        pltpu.async_copy(pe2.at[idx_slice(s)], pe_bufs[s], gsems[s])
    for s in range(NBUF - 1):
        pltpu.async_copy(row_slice(x_hbm, s), x_bufs[s], xsems[s])

    n_groups = (n_chunks + NBUF) // NBUF

    def group_body(g, carry):
        for b in range(NBUF):
            ci = g * NBUF + b

            @pl.when(ci < n_chunks)
            def _():
                drain(pe_bufs[b], gsems[b])
                drain(x_bufs[b], xsems[b])
                x_b, pe_b = x_bufs[b], pe_bufs[b]
                for r in range(CHUNK):
                    @plsc.parallel_loop(0, hidden, step=LANES, unroll=16)
                    def _(i):
                        seg = pl.ds(pl.multiple_of(i, LANES), LANES)
                        plsc.addupdate(x_b.at[r, seg], pe_b[r, seg])
                pltpu.async_copy(x_b, row_slice(out_hbm, ci), osems[b])
                @pl.when(ci + NBUF < n_chunks)
                def _():
                    pltpu.async_copy(
                        pe2.at[idx_slice(ci + NBUF)], pe_b, gsems[b])

            d = (b + 2) % NBUF

            @pl.when(jnp.logical_and(ci >= 1, ci - 1 < n_chunks))
            def _():
                drain(x_bufs[d], osems[d])

            @pl.when(ci + 2 < n_chunks)
            def _():
                pltpu.async_copy(
                    row_slice(x_hbm, ci + 2), x_bufs[d], xsems[d])
        return carry

    lax.fori_loop(0, n_groups, group_body, 0)


def kernel(x, spans, pe):
    batch, seq, hidden = x.shape
    assert NUM_SUBCORES * (R0 + R1) == batch * seq
    mesh = plsc.VectorSubcoreMesh(
        core_axis_name="c", subcore_axis_name="s",
        num_cores=NUM_CORES, num_subcores=NUM_SUBCORES)
    return pl.kernel(
        _body,
        out_type=jax.ShapeDtypeStruct((batch, seq, hidden), jnp.float32),
        mesh=mesh,
        scratch_types=(
            [pltpu.VMEM((max(R0, R1),), jnp.int32)]
            + [pltpu.VMEM((CHUNK, hidden), jnp.float32)] * (2 * NBUF)
            + [pltpu.SemaphoreType.DMA] * (3 * NBUF)
        ),
    )(x, spans.reshape(batch * seq).astype(jnp.int32), pe)


# speedup vs baseline: 1.0647x; 1.0642x over previous
import jax
import jax.numpy as jnp
from jax import lax
from jax.experimental import pallas as pl
from jax.experimental.pallas import tpu as pltpu
from jax.experimental.pallas import tpu_sc as plsc

NUM_CORES = 2
NUM_SUBCORES = 16
LANES = 16
NUM_WORKERS = NUM_CORES * NUM_SUBCORES

CHUNK = 8
NBUF = 3


def _body(x_hbm, spans_hbm, pe_hbm, out_hbm, idx_v, *scratch):
    x_bufs = scratch[:NBUF]
    pe_bufs = scratch[NBUF:2 * NBUF]
    gsems = scratch[2 * NBUF:3 * NBUF]
    xsems = scratch[3 * NBUF:4 * NBUF]
    osems = scratch[4 * NBUF:5 * NBUF]

    batch, seq, hidden = x_hbm.shape
    rows_per_w = (batch * seq) // NUM_WORKERS
    n_chunks = rows_per_w // CHUNK
    w_per_batch = seq // rows_per_w

    wid = lax.axis_index("s") * NUM_CORES + lax.axis_index("c")
    b_ix = wid // w_per_batch
    base = (wid % w_per_batch) * rows_per_w
    pe2 = pe_hbm.at[0]

    def idx_slice(ci):
        return idx_v.at[pl.ds(pl.multiple_of(ci * CHUNK, CHUNK), CHUNK)]

    def row_slice(hbm, ci):
        return hbm.at[b_ix, pl.ds(base + ci * CHUNK, CHUNK)]

    def drain(dst, sem):
        pltpu.make_async_copy(x_hbm.at[0, pl.ds(0, CHUNK)], dst, sem).wait()

    pltpu.sync_copy(spans_hbm.at[b_ix, pl.ds(base, rows_per_w)], idx_v)

    for s in range(NBUF):
        pltpu.async_copy(pe2.at[idx_slice(s)], pe_bufs[s], gsems[s])
    for s in range(NBUF - 1):
        pltpu.async_copy(row_slice(x_hbm, s), x_bufs[s], xsems[s])

    n_groups = (n_chunks + NBUF) // NBUF

    def group_body(g, carry):
        for b in range(NBUF):
            ci = g * NBUF + b

            @pl.when(ci < n_chunks)
            def _():
                drain(pe_bufs[b], gsems[b])
                drain(x_bufs[b], xsems[b])
                x_b, pe_b = x_bufs[b], pe_bufs[b]
                for r in range(CHUNK):
                    @plsc.parallel_loop(0, hidden, step=LANES, unroll=16)
                    def _(i):
                        seg = pl.ds(pl.multiple_of(i, LANES), LANES)
                        plsc.addupdate(x_b.at[r, seg], pe_b[r, seg])
                pltpu.async_copy(x_b, row_slice(out_hbm, ci), osems[b])
                @pl.when(ci + NBUF < n_chunks)
                def _():
                    pltpu.async_copy(
                        pe2.at[idx_slice(ci + NBUF)], pe_b, gsems[b])

            d = (b + 2) % NBUF

            @pl.when(jnp.logical_and(ci >= 1, ci - 1 < n_chunks))
            def _():
                drain(x_bufs[d], osems[d])

            @pl.when(ci + 2 < n_chunks)
            def _():
                pltpu.async_copy(
                    row_slice(x_hbm, ci + 2), x_bufs[d], xsems[d])
        return carry

    lax.fori_loop(0, n_groups, group_body, 0)


def kernel(x, spans, pe):
    batch, seq, hidden = x.shape
    rows_per_w = (batch * seq) // NUM_WORKERS
    mesh = plsc.VectorSubcoreMesh(
        core_axis_name="c", subcore_axis_name="s",
        num_cores=NUM_CORES, num_subcores=NUM_SUBCORES)
    return pl.kernel(
        _body,
        out_type=jax.ShapeDtypeStruct((batch, seq, hidden), jnp.float32),
        mesh=mesh,
        scratch_types=(
            [pltpu.VMEM((rows_per_w,), jnp.int32)]
            + [pltpu.VMEM((CHUNK, hidden), jnp.float32)] * (2 * NBUF)
            + [pltpu.SemaphoreType.DMA] * (3 * NBUF)
        ),
    )(x, spans.astype(jnp.int32), pe)
